# Initial kernel scaffold; baseline (speedup 1.0000x reference)
import jax
import jax.numpy as jnp
from jax.experimental import pallas as pl


def kernel(x, edge_index, batch, W_enc, b_enc, W1, b1, W2, b2, bn_gamma, bn_beta, vn0, Wv1, bv1, Wv2, bv2):
    raise NotImplementedError("write your pallas kernel here")


# speedup vs baseline: 2.9317x; 2.9317x over previous
import functools

import jax
import jax.numpy as jnp
from jax import lax
from jax.experimental import pallas as pl
from jax.experimental.pallas import tpu as pltpu
from jax.experimental.pallas import tpu_sc as plsc

N, E, H, G, LYR = 10000, 320000, 128, 64, 3
NC, NS = 2, 16
NW = NC * NS
EROWS_PT = 80
ER = NW * EROWS_PT
E_PAD = ER * 128
NPAD = 10240
ZR = NPAD // NS
OUT_PT = 632

_HIGH = lax.Precision.HIGHEST


def _mm(a, b):
    return lax.dot(a, b, precision=_HIGH, preferred_element_type=jnp.float32)


def _sc_aggr_body(r_hbm, src_hbm, dst_hbm, out_hbm,
                  src_v, dst_v, rows_v, zb, acc, sem):
    cid = lax.axis_index("c")
    sid = lax.axis_index("s")
    wid = sid * NC + cid

    for i in range(8):
        for k in range(H // 16):
            zb[i, pl.ds(k * 16, 16)] = jnp.zeros((16,), jnp.float32)

    @pl.loop(0, ZR // 8)
    def _zero(j):
        off = pl.multiple_of(sid * ZR + j * 8, 8)
        pltpu.sync_copy(zb, acc.at[pl.ds(off, 8)])

    eoff = pl.multiple_of(wid * EROWS_PT, 8)
    pltpu.sync_copy(src_hbm.at[pl.ds(eoff, EROWS_PT)], src_v)
    pltpu.sync_copy(dst_hbm.at[pl.ds(eoff, EROWS_PT)], dst_v)

    plsc.subcore_barrier()

    @pl.loop(0, EROWS_PT)
    def _edges(j):
        pltpu.async_copy(r_hbm.at[src_v.at[j]], rows_v, sem).wait()
        pltpu.sync_copy(rows_v, acc.at[dst_v.at[j]], add=True)

    plsc.subcore_barrier()

    @pl.when(sid < NS - 1)
    def _copy_main():
        ooff = pl.multiple_of(sid * OUT_PT, 8)
        pltpu.sync_copy(acc.at[pl.ds(ooff, OUT_PT)],
                        out_hbm.at[cid].at[pl.ds(ooff, OUT_PT)])

    @pl.when(sid == NS - 1)
    def _copy_tail():
        base = (NS - 1) * OUT_PT
        pltpu.sync_copy(acc.at[pl.ds(base, N - base)],
                        out_hbm.at[cid].at[pl.ds(base, N - base)])


@functools.lru_cache(maxsize=None)
def _sc_aggregate_fn():
    return pl.kernel(
        _sc_aggr_body,
        out_type=jax.ShapeDtypeStruct((NC, N, H), jnp.float32),
        mesh=plsc.VectorSubcoreMesh(core_axis_name="c", subcore_axis_name="s",
                                    num_cores=NC, num_subcores=NS),
        scratch_types=[
            pltpu.VMEM((EROWS_PT, 128), jnp.int32),
            pltpu.VMEM((EROWS_PT, 128), jnp.int32),
            pltpu.VMEM((128, H), jnp.float32),
            pltpu.VMEM((8, H), jnp.float32),
            pltpu.VMEM_SHARED((NPAD, H), jnp.float32),
            pltpu.SemaphoreType.DMA,
        ],
    )


def _sc_aggregate(r, src2d, dst2d):
    return _sc_aggregate_fn()(r, src2d, dst2d)


def _tc0_body(x, We, be, vn0r, hb_ref, r_ref):
    h = _mm(x[...], We[...]) + be[...]
    hb = h + vn0r[...]
    hb_ref[...] = hb
    r_ref[...] = jnp.maximum(hb, 0.0)


def _tc0(x, We, be, vn0r):
    return pl.pallas_call(
        _tc0_body,
        out_shape=[jax.ShapeDtypeStruct((N, H), jnp.float32)] * 2,
    )(x, We, be, vn0r)


def _tc_layer_body(hb, parts, batch2, batchT, vn,
                   W1l, b1l, W2l, b2l, gl, bl, Wv1l, bv1l, Wv2l, bv2l,
                   hb_new_ref, r_new_ref, vn_new_ref):
    hbv = hb[...]
    pv = parts[...]
    z0 = hbv + pv[0] + pv[1]
    z = _mm(jnp.maximum(_mm(z0, W1l[...]) + b1l[...], 0.0), W2l[...]) + b2l[...]
    z = gl[...] * z + bl[...]
    z = jnp.maximum(z, 0.0)
    h_out = z + hbv
    onehotT = (lax.broadcasted_iota(jnp.int32, (G, N), 0)
               == batchT[...]).astype(jnp.float32)
    s = _mm(onehotT, h_out)
    vt = jnp.maximum(_mm(s + vn[...], Wv1l[...]) + bv1l[...], 0.0)
    vn_new = jnp.maximum(_mm(vt, Wv2l[...]) + bv2l[...], 0.0)
    vn_new_ref[...] = vn_new
    onehot = (batch2[...] == lax.broadcasted_iota(jnp.int32, (N, G), 1)
              ).astype(jnp.float32)
    hb_new = h_out + _mm(onehot, vn_new)
    hb_new_ref[...] = hb_new
    r_new_ref[...] = jnp.maximum(hb_new, 0.0)


def _tc_layer(hb, parts, batch2, batchT, vn, W1l, b1l, W2l, b2l, gl, bl,
              Wv1l, bv1l, Wv2l, bv2l):
    return pl.pallas_call(
        _tc_layer_body,
        out_shape=[
            jax.ShapeDtypeStruct((N, H), jnp.float32),
            jax.ShapeDtypeStruct((N, H), jnp.float32),
            jax.ShapeDtypeStruct((G, H), jnp.float32),
        ],
    )(hb, parts, batch2, batchT, vn, W1l, b1l, W2l, b2l, gl, bl,
      Wv1l, bv1l, Wv2l, bv2l)


def _tc_final_body(hb, parts, batch2, W1l, b1l, W2l, b2l, gl, bl,
                   hrep_ref, hnode_ref):
    hbv = hb[...]
    pv = parts[...]
    z0 = hbv + pv[0] + pv[1]
    z = _mm(jnp.maximum(_mm(z0, W1l[...]) + b1l[...], 0.0), W2l[...]) + b2l[...]
    z = gl[...] * z + bl[...]
    h3 = z + hbv
    hnode_ref[...] = h3
    b2v = batch2[...]

    def body(g, carry):
        m = jnp.max(jnp.where(b2v == g, h3, -jnp.inf), axis=0, keepdims=True)
        hrep_ref[pl.ds(g, 1), :] = m
        return carry

    lax.fori_loop(0, G, body, 0)


def _tc_final(hb, parts, batch2, W1l, b1l, W2l, b2l, gl, bl):
    return pl.pallas_call(
        _tc_final_body,
        out_shape=[
            jax.ShapeDtypeStruct((G, H), jnp.float32),
            jax.ShapeDtypeStruct((N, H), jnp.float32),
        ],
    )(hb, parts, batch2, W1l, b1l, W2l, b2l, gl, bl)


def kernel(x, edge_index, batch, W_enc, b_enc, W1, b1, W2, b2,
           bn_gamma, bn_beta, vn0, Wv1, bv1, Wv2, bv2):
    batch2 = batch.reshape(N, 1)
    batchT = batch.reshape(1, N)
    be = b_enc.reshape(1, H)
    vn0r = vn0.reshape(1, H)

    src = edge_index[0]
    dst = edge_index[1]
    pad = E_PAD - E
    src2d = jnp.concatenate(
        [src, jnp.zeros((pad,), jnp.int32)]).reshape(ER, 128)
    dst2d = jnp.concatenate(
        [dst, jnp.full((pad,), N, jnp.int32)]).reshape(ER, 128)

    hb, r = _tc0(x, W_enc, be, vn0r)
    vn = jnp.broadcast_to(vn0r, (G, H))

    for l in range(LYR - 1):
        parts = _sc_aggregate(r, src2d, dst2d)
        hb, r, vn = _tc_layer(
            hb, parts, batch2, batchT, vn,
            W1[l], b1[l].reshape(1, -1), W2[l], b2[l].reshape(1, -1),
            bn_gamma[l].reshape(1, -1), bn_beta[l].reshape(1, -1),
            Wv1[l], bv1[l].reshape(1, -1), Wv2[l], bv2[l].reshape(1, -1))

    parts = _sc_aggregate(r, src2d, dst2d)
    l = LYR - 1
    h_rep, h_node = _tc_final(
        hb, parts, batch2,
        W1[l], b1[l].reshape(1, -1), W2[l], b2[l].reshape(1, -1),
        bn_gamma[l].reshape(1, -1), bn_beta[l].reshape(1, -1))
    return h_rep, h_node


# speedup vs baseline: 3.2669x; 1.1143x over previous
import functools

import jax
import jax.numpy as jnp
from jax import lax
from jax.experimental import pallas as pl
from jax.experimental.pallas import tpu as pltpu
from jax.experimental.pallas import tpu_sc as plsc

N, E, H, G, LYR = 10000, 320000, 128, 64, 3
NC, NS = 2, 16
NW = NC * NS
EROWS_PT = 80
ER = NW * EROWS_PT
E_PAD = ER * 128
NPAD = 10240
ZR = NPAD // NS
OUT_PT = 632
NBUF = 2
IDXH = EROWS_PT // 2

_HIGH = lax.Precision.HIGHEST


def _mm(a, b):
    return lax.dot(a, b, precision=_HIGH, preferred_element_type=jnp.float32)


def _sc_aggr_body(r_hbm, src_hbm, dst_hbm, out_hbm,
                  src_v, dst_v, rows_v, acc, gsem, ssem):
    cid = lax.axis_index("c")
    sid = lax.axis_index("s")
    wid = sid * NC + cid

    @pl.loop(0, 128)
    def _zrow(j):
        for b in range(NBUF):
            for k in range(H // 16):
                rows_v[b, j, pl.ds(k * 16, 16)] = jnp.zeros((16,), jnp.float32)

    @pl.loop(0, ZR // 128)
    def _zero(j):
        off = pl.multiple_of(sid * ZR + j * 128, 8)
        pltpu.sync_copy(rows_v.at[0], acc.at[pl.ds(off, 128)])

    plsc.subcore_barrier()

    for half in range(2):
        eoff = pl.multiple_of(wid * EROWS_PT + half * IDXH, 8)
        pltpu.sync_copy(src_hbm.at[pl.ds(eoff, IDXH)], src_v)
        pltpu.sync_copy(dst_hbm.at[pl.ds(eoff, IDXH)], dst_v)

        for b in range(NBUF):
            pltpu.async_copy(r_hbm.at[src_v.at[b]], rows_v.at[b], gsem)

        ngrp = IDXH // NBUF

        @pl.loop(0, ngrp)
        def _grp(g):
            j0 = g * NBUF
            for b in range(NBUF):
                pltpu.make_async_copy(r_hbm.at[pl.ds(0, 128)], rows_v.at[b],
                                      gsem).wait()
                pltpu.async_copy(rows_v.at[b], acc.at[dst_v.at[j0 + b]], ssem,
                                 add=True)
            for b in range(NBUF):
                pltpu.make_async_copy(rows_v.at[b], acc.at[pl.ds(0, 128)],
                                      ssem).wait()

                @pl.when(g < ngrp - 1)
                def _next():
                    pltpu.async_copy(r_hbm.at[src_v.at[j0 + NBUF + b]],
                                     rows_v.at[b], gsem)

    plsc.subcore_barrier()

    @pl.when(sid < NS - 1)
    def _copy_main():
        ooff = pl.multiple_of(sid * OUT_PT, 8)
        pltpu.sync_copy(acc.at[pl.ds(ooff, OUT_PT)],
                        out_hbm.at[cid].at[pl.ds(ooff, OUT_PT)])

    @pl.when(sid == NS - 1)
    def _copy_tail():
        base = (NS - 1) * OUT_PT
        pltpu.sync_copy(acc.at[pl.ds(base, N - base)],
                        out_hbm.at[cid].at[pl.ds(base, N - base)])


@functools.lru_cache(maxsize=None)
def _sc_aggregate_fn():
    return pl.kernel(
        _sc_aggr_body,
        out_type=jax.ShapeDtypeStruct((NC, N, H), jnp.float32),
        mesh=plsc.VectorSubcoreMesh(core_axis_name="c", subcore_axis_name="s",
                                    num_cores=NC, num_subcores=NS),
        scratch_types=[
            pltpu.VMEM((IDXH, 128), jnp.int32),
            pltpu.VMEM((IDXH, 128), jnp.int32),
            pltpu.VMEM((NBUF, 128, H), jnp.float32),
            pltpu.VMEM_SHARED((NPAD, H), jnp.float32),
            pltpu.SemaphoreType.DMA,
            pltpu.SemaphoreType.DMA,
        ],
    )


def _sc_aggregate(r, src2d, dst2d):
    return _sc_aggregate_fn()(r, src2d, dst2d)


def _tc0_body(x, We, be, vn0r, hb_ref, r_ref):
    h = _mm(x[...], We[...]) + be[...]
    hb = h + vn0r[...]
    hb_ref[...] = hb
    r_ref[...] = jnp.maximum(hb, 0.0)


def _tc0(x, We, be, vn0r):
    return pl.pallas_call(
        _tc0_body,
        out_shape=[jax.ShapeDtypeStruct((N, H), jnp.float32)] * 2,
    )(x, We, be, vn0r)


def _tc_layer_body(hb, parts, batch2, batchT, vn,
                   W1l, b1l, W2l, b2l, gl, bl, Wv1l, bv1l, Wv2l, bv2l,
                   hb_new_ref, r_new_ref, vn_new_ref):
    hbv = hb[...]
    pv = parts[...]
    z0 = hbv + pv[0] + pv[1]
    z = _mm(jnp.maximum(_mm(z0, W1l[...]) + b1l[...], 0.0), W2l[...]) + b2l[...]
    z = gl[...] * z + bl[...]
    z = jnp.maximum(z, 0.0)
    h_out = z + hbv
    onehotT = (lax.broadcasted_iota(jnp.int32, (G, N), 0)
               == batchT[...]).astype(jnp.float32)
    s = _mm(onehotT, h_out)
    vt = jnp.maximum(_mm(s + vn[...], Wv1l[...]) + bv1l[...], 0.0)
    vn_new = jnp.maximum(_mm(vt, Wv2l[...]) + bv2l[...], 0.0)
    vn_new_ref[...] = vn_new
    onehot = (batch2[...] == lax.broadcasted_iota(jnp.int32, (N, G), 1)
              ).astype(jnp.float32)
    hb_new = h_out + _mm(onehot, vn_new)
    hb_new_ref[...] = hb_new
    r_new_ref[...] = jnp.maximum(hb_new, 0.0)


def _tc_layer(hb, parts, batch2, batchT, vn, W1l, b1l, W2l, b2l, gl, bl,
              Wv1l, bv1l, Wv2l, bv2l):
    return pl.pallas_call(
        _tc_layer_body,
        out_shape=[
            jax.ShapeDtypeStruct((N, H), jnp.float32),
            jax.ShapeDtypeStruct((N, H), jnp.float32),
            jax.ShapeDtypeStruct((G, H), jnp.float32),
        ],
    )(hb, parts, batch2, batchT, vn, W1l, b1l, W2l, b2l, gl, bl,
      Wv1l, bv1l, Wv2l, bv2l)


def _tc_final_body(hb, parts, batch2, W1l, b1l, W2l, b2l, gl, bl,
                   hrep_ref, hnode_ref):
    hbv = hb[...]
    pv = parts[...]
    z0 = hbv + pv[0] + pv[1]
    z = _mm(jnp.maximum(_mm(z0, W1l[...]) + b1l[...], 0.0), W2l[...]) + b2l[...]
    z = gl[...] * z + bl[...]
    h3 = z + hbv
    hnode_ref[...] = h3
    b2v = batch2[...]

    def body(g, carry):
        m = jnp.max(jnp.where(b2v == g, h3, -jnp.inf), axis=0, keepdims=True)
        hrep_ref[pl.ds(g, 1), :] = m
        return carry

    lax.fori_loop(0, G, body, 0)


def _tc_final(hb, parts, batch2, W1l, b1l, W2l, b2l, gl, bl):
    return pl.pallas_call(
        _tc_final_body,
        out_shape=[
            jax.ShapeDtypeStruct((G, H), jnp.float32),
            jax.ShapeDtypeStruct((N, H), jnp.float32),
        ],
    )(hb, parts, batch2, W1l, b1l, W2l, b2l, gl, bl)


def kernel(x, edge_index, batch, W_enc, b_enc, W1, b1, W2, b2,
           bn_gamma, bn_beta, vn0, Wv1, bv1, Wv2, bv2):
    batch2 = batch.reshape(N, 1)
    batchT = batch.reshape(1, N)
    be = b_enc.reshape(1, H)
    vn0r = vn0.reshape(1, H)

    src = edge_index[0]
    dst = edge_index[1]
    pad = E_PAD - E
    src2d = jnp.concatenate(
        [src, jnp.zeros((pad,), jnp.int32)]).reshape(ER, 128)
    dst2d = jnp.concatenate(
        [dst, jnp.full((pad,), N, jnp.int32)]).reshape(ER, 128)

    hb, r = _tc0(x, W_enc, be, vn0r)
    vn = jnp.broadcast_to(vn0r, (G, H))

    for l in range(LYR - 1):
        parts = _sc_aggregate(r, src2d, dst2d)
        hb, r, vn = _tc_layer(
            hb, parts, batch2, batchT, vn,
            W1[l], b1[l].reshape(1, -1), W2[l], b2[l].reshape(1, -1),
            bn_gamma[l].reshape(1, -1), bn_beta[l].reshape(1, -1),
            Wv1[l], bv1[l].reshape(1, -1), Wv2[l], bv2[l].reshape(1, -1))

    parts = _sc_aggregate(r, src2d, dst2d)
    l = LYR - 1
    h_rep, h_node = _tc_final(
        hb, parts, batch2,
        W1[l], b1[l].reshape(1, -1), W2[l], b2[l].reshape(1, -1),
        bn_gamma[l].reshape(1, -1), bn_beta[l].reshape(1, -1))
    return h_rep, h_node
